# Initial kernel scaffold; baseline (speedup 1.0000x reference)
import jax
import jax.numpy as jnp
from jax.experimental import pallas as pl


def kernel(x, edge_index, w, W1, W2, W3, g1, b1, g2, b2):
    raise NotImplementedError("write your pallas kernel here")


# speedup vs baseline: 1.0697x; 1.0697x over previous
import functools

import jax
import jax.numpy as jnp
from jax import lax
from jax.experimental import pallas as pl
from jax.experimental.pallas import tpu as pltpu
from jax.experimental.pallas import tpu_sc as plsc

NC = 2
NS = 16
NW = NC * NS

_N = 10000
_E = 320000
_D = 128

EPW = _E // NW
C = 80
G = EPW // C
RPT = _N // NS
FC = 125
NP = 10240
PW = NP // NS
WCH = 2000
NT = EPW // WCH

_mesh = plsc.VectorSubcoreMesh(core_axis_name="c", subcore_axis_name="s",
                               num_cores=NC, num_subcores=NS)
_sc_params = pltpu.CompilerParams(needs_layout_passes=False)


def _wid():
    return lax.axis_index("s") * NC + lax.axis_index("c")


def _deg_body(src_hbm, dst_hbm, out_hbm, idxs_v, idxd_v, ones_v, zer_v,
              stage_v, dsrc_sh, ddst_sh):
    c = lax.axis_index("c")
    s = lax.axis_index("s")
    wid = s * NC + c

    def setv(i, _):
        ones_v[pl.ds(i * 16, 16)] = jnp.full((16,), 1.0, jnp.float32)
        return 0
    lax.fori_loop(0, C // 16, setv, 0)

    def zv(i, _):
        zer_v[pl.ds(i * 16, 16)] = jnp.zeros((16,), jnp.float32)
        return 0
    lax.fori_loop(0, PW // 16, zv, 0)

    pltpu.sync_copy(zer_v, dsrc_sh.at[pl.ds(s * PW, PW)])
    pltpu.sync_copy(zer_v, ddst_sh.at[pl.ds(s * PW, PW)])
    plsc.subcore_barrier()

    pltpu.sync_copy(src_hbm.at[wid], idxs_v)
    pltpu.sync_copy(dst_hbm.at[wid], idxd_v)

    def body(g, _):
        pltpu.sync_copy(ones_v, dsrc_sh.at[idxs_v.at[g]], add=True)
        pltpu.sync_copy(ones_v, ddst_sh.at[idxd_v.at[g]], add=True)
        return 0
    lax.fori_loop(0, G, body, 0)
    plsc.subcore_barrier()

    pltpu.sync_copy(dsrc_sh.at[pl.ds(s * PW, PW)], stage_v)
    pltpu.sync_copy(stage_v, out_hbm.at[pl.ds(c * 2 * NP + s * PW, PW)])
    pltpu.sync_copy(ddst_sh.at[pl.ds(s * PW, PW)], stage_v)
    pltpu.sync_copy(stage_v, out_hbm.at[pl.ds(c * 2 * NP + NP + s * PW, PW)])


_deg_call = functools.partial(
    pl.kernel, _deg_body,
    out_type=jax.ShapeDtypeStruct((2 * 2 * NP,), jnp.float32),
    mesh=_mesh,
    compiler_params=_sc_params,
    scratch_types=[
        pltpu.VMEM((G, C), jnp.int32),
        pltpu.VMEM((G, C), jnp.int32),
        pltpu.VMEM((C,), jnp.float32),
        pltpu.VMEM((PW,), jnp.float32),
        pltpu.VMEM((PW,), jnp.float32),
        pltpu.VMEM_SHARED((NP,), jnp.float32),
        pltpu.VMEM_SHARED((NP,), jnp.float32),
    ],
)


def _we_body(src_hbm, wf_hbm, dsq_hbm, out_hbm, src_v, dso_v, wf_v, we_v):
    wid = _wid()
    base = wid * EPW
    pltpu.sync_copy(src_hbm.at[pl.ds(base, EPW)], src_v)
    pltpu.sync_copy(dsq_hbm.at[0], dso_v)
    inv = jnp.float32(1.0 / 16.0)
    for t in range(NT):
        pltpu.sync_copy(
            wf_hbm.at[pl.ds((base + t * WCH) * 16, WCH * 16)], wf_v)

        def body(k, _):
            e0 = t * WCH + k * 16
            bidx = lax.iota(jnp.int32, 16) * 16 + k * 256
            acc = plsc.load_gather(wf_v, [bidx])
            for j in range(1, 16):
                acc = acc + plsc.load_gather(wf_v, [bidx + j])
            sv = src_v[pl.ds(e0, 16)]
            dg = plsc.load_gather(dso_v, [sv])
            we_v[pl.ds(e0, 16)] = acc * inv * dg
            return 0
        lax.fori_loop(0, WCH // 16, body, 0)
    pltpu.sync_copy(we_v, out_hbm.at[pl.ds(base, EPW)])


_we_call = functools.partial(
    pl.kernel, _we_body,
    out_type=jax.ShapeDtypeStruct((_E,), jnp.float32),
    mesh=_mesh,
    compiler_params=_sc_params,
    scratch_types=[
        pltpu.VMEM((EPW,), jnp.int32),
        pltpu.VMEM((NP,), jnp.float32),
        pltpu.VMEM((WCH * 16,), jnp.float32),
        pltpu.VMEM((EPW,), jnp.float32),
    ],
)


DH = _D // 2
EPC = C * DH
NDESC = EPC // 128
FLW = 2560
FR = NP * DH // NS // FLW


def _mp_body(half, h_hbm, src_hbm, dst_hbm, we_hbm, out_hbm,
             idxs_v, idxd_v, wec_v, rows_v, rowsh_v, idxb_v, acc_sh,
             sem, sem2):
    c = lax.axis_index("c")
    s = lax.axis_index("s")
    wid = s * NC + c

    def zb(i, _):
        rowsh_v[pl.ds(i * 16, 16)] = jnp.zeros((16,), jnp.float32)
        return 0
    lax.fori_loop(0, FLW // 16, zb, 0)
    for k in range(FR):
        pltpu.sync_copy(rowsh_v.at[pl.ds(0, FLW)],
                        acc_sh.at[pl.ds((s * FR + k) * FLW, FLW)])
    plsc.subcore_barrier()

    base = wid * EPW
    ioff = [lax.iota(jnp.int32, 16) + j * 16 for j in range(DH // 16)]

    def chunk(g, _):
        off = base + g * C
        pltpu.sync_copy(src_hbm.at[pl.ds(off, C)], idxs_v)
        pltpu.sync_copy(we_hbm.at[pl.ds(off, C)], wec_v)
        pltpu.sync_copy(dst_hbm.at[pl.ds(off, C)], idxd_v)
        pltpu.async_copy(h_hbm.at[idxs_v], rows_v, sem).wait()

        def medge(i, _):
            lane16 = jnp.full((16,), i, jnp.int32)
            wsc = plsc.load_gather(wec_v, [lane16])
            dv = plsc.load_gather(idxd_v, [lane16]) * 64
            r = i // 2
            c0 = (i % 2) * DH
            for j in range(DH // 16):
                rowsh_v[pl.ds(i * DH + j * 16, 16)] = (
                    rows_v[i, pl.ds(half * DH + j * 16, 16)] * wsc)
                idxb_v[r, pl.ds(c0 + j * 16, 16)] = dv + ioff[j]
            return 0
        lax.fori_loop(0, C, medge, 0)

        descs = []
        for r in range(NDESC):
            descs.append(pltpu.async_copy(
                rowsh_v.at[pl.ds(r * 128, 128)],
                acc_sh.at[idxb_v.at[r]], sem2, add=True))
        for d in descs:
            d.wait()
        return 0
    lax.fori_loop(0, G, chunk, 0)
    plsc.subcore_barrier()

    for k in range(FR):
        r0 = (s * FR + k) * FLW
        pltpu.sync_copy(acc_sh.at[pl.ds(r0, FLW)],
                        rowsh_v.at[pl.ds(0, FLW)])
        pltpu.sync_copy(rowsh_v.at[pl.ds(0, FLW)],
                        out_hbm.at[pl.ds(c * NP * DH + r0, FLW)])


def _mp_call(half):
    return pl.kernel(
        functools.partial(_mp_body, half),
        out_type=jax.ShapeDtypeStruct((2 * NP * DH,), jnp.float32),
        mesh=_mesh,
        compiler_params=_sc_params,
        scratch_types=[
            pltpu.VMEM((C,), jnp.int32),
            pltpu.VMEM((C,), jnp.int32),
            pltpu.VMEM((C,), jnp.float32),
            pltpu.VMEM((C, _D), jnp.float32),
            pltpu.VMEM((EPC,), jnp.float32),
            pltpu.VMEM((NDESC, 128), jnp.int32),
            pltpu.VMEM_SHARED((NP * DH,), jnp.float32),
            pltpu.SemaphoreType.DMA,
            pltpu.SemaphoreType.DMA,
        ],
    )


def _dsq_body(p_ref, o_ref):
    a = p_ref[0:2, :] + p_ref[2:4, :]
    o_ref[...] = lax.rsqrt(jnp.maximum(a, 1.0))


def _dsq_call(degp):
    return pl.pallas_call(
        _dsq_body,
        out_shape=jax.ShapeDtypeStruct((2, NP), jnp.float32),
    )(degp)


_BN = 400
_GN = _N // _BN


def _dense_body(pa0_ref, pa1_ref, pb0_ref, pb1_ref, dsi_ref, w_ref,
                g_ref, b_ref, o_ref):
    agg = jnp.concatenate(
        [pa0_ref[...] + pa1_ref[...], pb0_ref[...] + pb1_ref[...]], axis=1)
    agg = agg * dsi_ref[...]
    h = jnp.dot(agg, w_ref[...], preferred_element_type=jnp.float32)
    h = jnp.maximum(h, 0.0)
    mu = jnp.mean(h, axis=-1, keepdims=True)
    var = jnp.mean((h - mu) ** 2, axis=-1, keepdims=True)
    o_ref[...] = (h - mu) * lax.rsqrt(var + 1e-5) * g_ref[...] + b_ref[...]


_half_specs = [
    pl.BlockSpec((_BN, DH), lambda i: (i, 0)),
    pl.BlockSpec((_BN, DH), lambda i: (i, 0)),
    pl.BlockSpec((_BN, DH), lambda i: (i, 0)),
    pl.BlockSpec((_BN, DH), lambda i: (i, 0)),
    pl.BlockSpec((_BN, 1), lambda i: (i, 0)),
]


def _dense_call(parts, dsi2, W, gg, bb):
    return pl.pallas_call(
        _dense_body,
        grid=(_GN,),
        in_specs=_half_specs + [
            pl.BlockSpec((_D, _D), lambda i: (0, 0)),
            pl.BlockSpec((1, _D), lambda i: (0, 0)),
            pl.BlockSpec((1, _D), lambda i: (0, 0)),
        ],
        out_specs=pl.BlockSpec((_BN, _D), lambda i: (i, 0)),
        out_shape=jax.ShapeDtypeStruct((_N, _D), jnp.float32),
    )(*parts, dsi2, W, gg.reshape(1, _D), bb.reshape(1, _D))


def _final_body(pa0_ref, pa1_ref, pb0_ref, pb1_ref, dsi_ref, w_ref, o_ref):
    i = pl.program_id(0)
    agg = jnp.concatenate(
        [pa0_ref[...] + pa1_ref[...], pb0_ref[...] + pb1_ref[...]], axis=1)
    agg = agg * dsi_ref[...]
    part = jnp.sum(agg, axis=0, keepdims=True)

    @pl.when(i == 0)
    def _():
        o_ref[...] = part

    @pl.when(i > 0)
    def _():
        o_ref[...] = o_ref[...] + part

    @pl.when(i == _GN - 1)
    def _():
        o_ref[...] = jnp.dot(o_ref[...], w_ref[...],
                             preferred_element_type=jnp.float32)


def _final_call(parts, dsi2, W):
    return pl.pallas_call(
        _final_body,
        grid=(_GN,),
        in_specs=_half_specs + [pl.BlockSpec((_D, _D), lambda i: (0, 0))],
        out_specs=pl.BlockSpec((1, _D), lambda i: (0, 0)),
        out_shape=jax.ShapeDtypeStruct((1, _D), jnp.float32),
    )(*parts, dsi2, W)


def kernel(x, edge_index, w, W1, W2, W3, g1, b1, g2, b2):
    src = edge_index[0]
    dst = edge_index[1]
    srcr = src.reshape(NW, G, C)
    dstr = dst.reshape(NW, G, C)
    wf = w.reshape(-1)

    degp = _deg_call()(srcr, dstr)
    dsq = _dsq_call(degp.reshape(4, NP))
    we = _we_call()(src, wf, dsq)
    wer = we.reshape(NW, G, C)
    dsi2 = dsq[1, :_N].reshape(_N, 1)

    h = x
    mpa = _mp_call(0)
    mpb = _mp_call(1)
    for li in range(3):
        pa = mpa(h, src, dst, we).reshape(2, NP, DH)
        pb = mpb(h, src, dst, we).reshape(2, NP, DH)
        parts = (pa[0, :_N], pa[1, :_N], pb[0, :_N], pb[1, :_N])
        if li < 2:
            Wl, ggl, bbl = ((W1, g1, b1), (W2, g2, b2))[li]
            h = _dense_call(parts, dsi2, Wl, ggl, bbl)
        else:
            out = _final_call(parts, dsi2, W3)
    return out


# speedup vs baseline: 1.2317x; 1.1515x over previous
import functools

import jax
import jax.numpy as jnp
from jax import lax
from jax.experimental import pallas as pl
from jax.experimental.pallas import tpu as pltpu
from jax.experimental.pallas import tpu_sc as plsc

NC = 2
NS = 16
NW = NC * NS

_N = 10000
_E = 320000
_D = 128

EPW = _E // NW
C = 80
G = EPW // C
RPT = _N // NS
FC = 125
NP = 10240
PW = NP // NS
WCH = 2000
NT = EPW // WCH

_mesh = plsc.VectorSubcoreMesh(core_axis_name="c", subcore_axis_name="s",
                               num_cores=NC, num_subcores=NS)
_sc_params = pltpu.CompilerParams(needs_layout_passes=False)


def _wid():
    return lax.axis_index("s") * NC + lax.axis_index("c")


def _deg_body(src_hbm, dst_hbm, out_hbm, idxs_v, idxd_v, ones_v, zer_v,
              stage_v, dsrc_sh, ddst_sh):
    c = lax.axis_index("c")
    s = lax.axis_index("s")
    wid = s * NC + c

    def setv(i, _):
        ones_v[pl.ds(i * 16, 16)] = jnp.full((16,), 1.0, jnp.float32)
        return 0
    lax.fori_loop(0, C // 16, setv, 0)

    def zv(i, _):
        zer_v[pl.ds(i * 16, 16)] = jnp.zeros((16,), jnp.float32)
        return 0
    lax.fori_loop(0, PW // 16, zv, 0)

    pltpu.sync_copy(zer_v, dsrc_sh.at[pl.ds(s * PW, PW)])
    pltpu.sync_copy(zer_v, ddst_sh.at[pl.ds(s * PW, PW)])
    plsc.subcore_barrier()

    pltpu.sync_copy(src_hbm.at[wid], idxs_v)
    pltpu.sync_copy(dst_hbm.at[wid], idxd_v)

    def body(g, _):
        pltpu.sync_copy(ones_v, dsrc_sh.at[idxs_v.at[g]], add=True)
        pltpu.sync_copy(ones_v, ddst_sh.at[idxd_v.at[g]], add=True)
        return 0
    lax.fori_loop(0, G, body, 0)
    plsc.subcore_barrier()

    pltpu.sync_copy(dsrc_sh.at[pl.ds(s * PW, PW)], stage_v)
    pltpu.sync_copy(stage_v, out_hbm.at[pl.ds(c * 2 * NP + s * PW, PW)])
    pltpu.sync_copy(ddst_sh.at[pl.ds(s * PW, PW)], stage_v)
    pltpu.sync_copy(stage_v, out_hbm.at[pl.ds(c * 2 * NP + NP + s * PW, PW)])


_deg_call = functools.partial(
    pl.kernel, _deg_body,
    out_type=jax.ShapeDtypeStruct((2 * 2 * NP,), jnp.float32),
    mesh=_mesh,
    compiler_params=_sc_params,
    scratch_types=[
        pltpu.VMEM((G, C), jnp.int32),
        pltpu.VMEM((G, C), jnp.int32),
        pltpu.VMEM((C,), jnp.float32),
        pltpu.VMEM((PW,), jnp.float32),
        pltpu.VMEM((PW,), jnp.float32),
        pltpu.VMEM_SHARED((NP,), jnp.float32),
        pltpu.VMEM_SHARED((NP,), jnp.float32),
    ],
)


def _we_body(src_hbm, wf_hbm, dsq_hbm, out_hbm, src_v, dso_v, wf_v, we_v):
    wid = _wid()
    base = wid * EPW
    pltpu.sync_copy(src_hbm.at[pl.ds(base, EPW)], src_v)
    pltpu.sync_copy(dsq_hbm.at[0], dso_v)
    inv = jnp.float32(1.0 / 16.0)
    for t in range(NT):
        pltpu.sync_copy(
            wf_hbm.at[pl.ds((base + t * WCH) * 16, WCH * 16)], wf_v)

        def body(k, _):
            e0 = t * WCH + k * 16
            bidx = lax.iota(jnp.int32, 16) * 16 + k * 256
            acc = plsc.load_gather(wf_v, [bidx])
            for j in range(1, 16):
                acc = acc + plsc.load_gather(wf_v, [bidx + j])
            sv = src_v[pl.ds(e0, 16)]
            dg = plsc.load_gather(dso_v, [sv])
            we_v[pl.ds(e0, 16)] = acc * inv * dg
            return 0
        lax.fori_loop(0, WCH // 16, body, 0)
    pltpu.sync_copy(we_v, out_hbm.at[pl.ds(base, EPW)])


_we_call = functools.partial(
    pl.kernel, _we_body,
    out_type=jax.ShapeDtypeStruct((_E,), jnp.float32),
    mesh=_mesh,
    compiler_params=_sc_params,
    scratch_types=[
        pltpu.VMEM((EPW,), jnp.int32),
        pltpu.VMEM((NP,), jnp.float32),
        pltpu.VMEM((WCH * 16,), jnp.float32),
        pltpu.VMEM((EPW,), jnp.float32),
    ],
)


DH = _D // 2
EPC = C * DH
NDESC = EPC // 128
FLW = 2560
FR = NP * DH // NS // FLW


def _mp_body(half, h_hbm, src_hbm, dst_hbm, we_hbm, out_hbm,
             idxs_v, idxd_v, wec_v, rows_v, rowsh_v, idxb_v, acc_sh,
             sem, sem2):
    c = lax.axis_index("c")
    s = lax.axis_index("s")
    wid = s * NC + c

    def zb(i, _):
        rowsh_v[pl.ds(i * 16, 16)] = jnp.zeros((16,), jnp.float32)
        return 0
    lax.fori_loop(0, FLW // 16, zb, 0)
    for k in range(FR):
        pltpu.sync_copy(rowsh_v.at[pl.ds(0, FLW)],
                        acc_sh.at[pl.ds((s * FR + k) * FLW, FLW)])
    plsc.subcore_barrier()

    base = wid * EPW
    ioff = [lax.iota(jnp.int32, 16) + j * 16 for j in range(DH // 16)]

    def chunk(g, _):
        off = base + g * C
        d1 = pltpu.async_copy(src_hbm.at[pl.ds(off, C)], idxs_v, sem)
        d2 = pltpu.async_copy(we_hbm.at[pl.ds(off, C)], wec_v, sem)
        d3 = pltpu.async_copy(dst_hbm.at[pl.ds(off, C)], idxd_v, sem)
        d1.wait()
        d2.wait()
        d3.wait()
        pltpu.async_copy(h_hbm.at[idxs_v], rows_v, sem).wait()

        def medge(i, _):
            lane16 = jnp.full((16,), i, jnp.int32)
            wsc = plsc.load_gather(wec_v, [lane16])
            dv = plsc.load_gather(idxd_v, [lane16]) * 64
            r = i // 2
            c0 = (i % 2) * DH
            for j in range(DH // 16):
                rowsh_v[pl.ds(i * DH + j * 16, 16)] = (
                    rows_v[i, pl.ds(half * DH + j * 16, 16)] * wsc)
                idxb_v[r, pl.ds(c0 + j * 16, 16)] = dv + ioff[j]
            return 0
        lax.fori_loop(0, C, medge, 0)

        descs = []
        for r in range(NDESC):
            descs.append(pltpu.async_copy(
                rowsh_v.at[pl.ds(r * 128, 128)],
                acc_sh.at[idxb_v.at[r]], sem2, add=True))
        for d in descs:
            d.wait()
        return 0
    lax.fori_loop(0, G, chunk, 0)
    plsc.subcore_barrier()

    for k in range(FR):
        r0 = (s * FR + k) * FLW
        pltpu.sync_copy(acc_sh.at[pl.ds(r0, FLW)],
                        rowsh_v.at[pl.ds(0, FLW)])
        pltpu.sync_copy(rowsh_v.at[pl.ds(0, FLW)],
                        out_hbm.at[pl.ds(c * NP * DH + r0, FLW)])


def _mp_call(half):
    return pl.kernel(
        functools.partial(_mp_body, half),
        out_type=jax.ShapeDtypeStruct((2 * NP * DH,), jnp.float32),
        mesh=_mesh,
        compiler_params=_sc_params,
        scratch_types=[
            pltpu.VMEM((C,), jnp.int32),
            pltpu.VMEM((C,), jnp.int32),
            pltpu.VMEM((C,), jnp.float32),
            pltpu.VMEM((C, _D), jnp.float32),
            pltpu.VMEM((EPC,), jnp.float32),
            pltpu.VMEM((NDESC, 128), jnp.int32),
            pltpu.VMEM_SHARED((NP * DH,), jnp.float32),
            pltpu.SemaphoreType.DMA,
            pltpu.SemaphoreType.DMA,
        ],
    )


def _dsq_body(p_ref, o_ref):
    a = p_ref[0:2, :] + p_ref[2:4, :]
    o_ref[...] = lax.rsqrt(jnp.maximum(a, 1.0))


def _dsq_call(degp):
    return pl.pallas_call(
        _dsq_body,
        out_shape=jax.ShapeDtypeStruct((2, NP), jnp.float32),
    )(degp)


_BN = 400
_GN = _N // _BN


def _dense_body(pa0_ref, pa1_ref, pb0_ref, pb1_ref, dsi_ref, w_ref,
                g_ref, b_ref, o_ref):
    agg = jnp.concatenate(
        [pa0_ref[...] + pa1_ref[...], pb0_ref[...] + pb1_ref[...]], axis=1)
    agg = agg * dsi_ref[...]
    h = jnp.dot(agg, w_ref[...], preferred_element_type=jnp.float32)
    h = jnp.maximum(h, 0.0)
    mu = jnp.mean(h, axis=-1, keepdims=True)
    var = jnp.mean((h - mu) ** 2, axis=-1, keepdims=True)
    o_ref[...] = (h - mu) * lax.rsqrt(var + 1e-5) * g_ref[...] + b_ref[...]


_half_specs = [
    pl.BlockSpec((_BN, DH), lambda i: (i, 0)),
    pl.BlockSpec((_BN, DH), lambda i: (i, 0)),
    pl.BlockSpec((_BN, DH), lambda i: (i, 0)),
    pl.BlockSpec((_BN, DH), lambda i: (i, 0)),
    pl.BlockSpec((_BN, 1), lambda i: (i, 0)),
]


def _dense_call(parts, dsi2, W, gg, bb):
    return pl.pallas_call(
        _dense_body,
        grid=(_GN,),
        in_specs=_half_specs + [
            pl.BlockSpec((_D, _D), lambda i: (0, 0)),
            pl.BlockSpec((1, _D), lambda i: (0, 0)),
            pl.BlockSpec((1, _D), lambda i: (0, 0)),
        ],
        out_specs=pl.BlockSpec((_BN, _D), lambda i: (i, 0)),
        out_shape=jax.ShapeDtypeStruct((_N, _D), jnp.float32),
    )(*parts, dsi2, W, gg.reshape(1, _D), bb.reshape(1, _D))


def _final_body(pa0_ref, pa1_ref, pb0_ref, pb1_ref, dsi_ref, w_ref, o_ref):
    i = pl.program_id(0)
    agg = jnp.concatenate(
        [pa0_ref[...] + pa1_ref[...], pb0_ref[...] + pb1_ref[...]], axis=1)
    agg = agg * dsi_ref[...]
    part = jnp.sum(agg, axis=0, keepdims=True)

    @pl.when(i == 0)
    def _():
        o_ref[...] = part

    @pl.when(i > 0)
    def _():
        o_ref[...] = o_ref[...] + part

    @pl.when(i == _GN - 1)
    def _():
        o_ref[...] = jnp.dot(o_ref[...], w_ref[...],
                             preferred_element_type=jnp.float32)


def _final_call(parts, dsi2, W):
    return pl.pallas_call(
        _final_body,
        grid=(_GN,),
        in_specs=_half_specs + [pl.BlockSpec((_D, _D), lambda i: (0, 0))],
        out_specs=pl.BlockSpec((1, _D), lambda i: (0, 0)),
        out_shape=jax.ShapeDtypeStruct((1, _D), jnp.float32),
    )(*parts, dsi2, W)


def kernel(x, edge_index, w, W1, W2, W3, g1, b1, g2, b2):
    src = edge_index[0]
    dst = edge_index[1]
    srcr = src.reshape(NW, G, C)
    dstr = dst.reshape(NW, G, C)
    wf = w.reshape(-1)

    degp = _deg_call()(srcr, dstr)
    dsq = _dsq_call(degp.reshape(4, NP))
    we = _we_call()(src, wf, dsq)
    wer = we.reshape(NW, G, C)
    dsi2 = dsq[1, :_N].reshape(_N, 1)

    h = x
    mpa = _mp_call(0)
    mpb = _mp_call(1)
    for li in range(3):
        pa = mpa(h, src, dst, we).reshape(2, NP, DH)
        pb = mpb(h, src, dst, we).reshape(2, NP, DH)
        parts = (pa[0, :_N], pa[1, :_N], pb[0, :_N], pb[1, :_N])
        if li < 2:
            Wl, ggl, bbl = ((W1, g1, b1), (W2, g2, b2))[li]
            h = _dense_call(parts, dsi2, Wl, ggl, bbl)
        else:
            out = _final_call(parts, dsi2, W3)
    return out
